# Initial kernel scaffold; baseline (speedup 1.0000x reference)
import jax
import jax.numpy as jnp
from jax.experimental import pallas as pl


def kernel(scribbles, outputs):
    raise NotImplementedError("write your pallas kernel here")


# speedup vs baseline: 3.5176x; 3.5176x over previous
import functools

import jax
import jax.numpy as jnp
from jax.experimental import pallas as pl
from jax.experimental.pallas import tpu as pltpu

_B, _C, _H, _W = 8, 16, 512, 512
_HB = 128
_NH = _H // _HB
_DELTA = 10.0
_NS_GT0 = 60.0
_NS_EQ0 = 10.0


def _lane_or(x):
    r = x[:, 0:128] | x[:, 128:256] | x[:, 256:384] | x[:, 384:512]
    for sh in (64, 32, 16, 8, 4, 2, 1):
        r = r | pltpu.roll(r, sh, axis=1)
    return r[:, 0:1]


def _sublane_or(x):
    r = x[0:32] | x[32:64] | x[64:96] | x[96:128]
    r = r[0:8] | r[8:16] | r[16:24] | r[24:32]
    for sh in (4, 2, 1):
        r = r | pltpu.roll(r, sh, axis=0)
    return r[0:1]


def _axis_stats(s, y, red_axis, idx, limit):
    kd = dict(axis=red_axis, keepdims=True)
    n1 = jnp.sum(s, **kd)
    d_pos = jnp.sum(s * (1.0 - y), **kd) / _NS_GT0
    d_neg = jnp.sum((1.0 - s) * y, **kd) / _NS_EQ0
    cs = jnp.minimum(1.0 / (1.0 + d_pos), 1.0 / (1.0 + d_neg))
    ext = jnp.floor(cs * _NS_EQ0)

    io0 = jax.lax.broadcasted_iota(jnp.int32, (_W, _W), 0)
    io1 = jax.lax.broadcasted_iota(jnp.int32, (_W, _W), 1)
    if red_axis == 1:
        tri = jnp.where(io0 <= io1, 1.0, 0.0)
        cums = jnp.dot(s, tri, preferred_element_type=jnp.float32)
    else:
        tri = jnp.where(io0 >= io1, 1.0, 0.0)
        cums = jnp.dot(tri, s, preferred_element_type=jnp.float32)

    so = n1 - cums
    sz = (float(_W - 1) - idx) - so
    one = jnp.float32(1.0)
    zero = jnp.float32(0.0)
    valid_bp = jnp.where(ext > 0.0, one, zero) * jnp.where(n1 > 0.0, one, zero)
    sel = jnp.where(
        s > 0.0,
        jnp.where(so < ext, one, zero),
        jnp.where(sz + n1 < ext, one, zero),
    )
    p = jnp.maximum(sel * valid_bp, y)

    any_p = jnp.max(p, **kd)
    lo = jnp.min(jnp.where(p > 0.0, idx, float(limit)), **kd)
    hi = jnp.max(jnp.where(p > 0.0, idx, -1.0), **kd)
    lo = jnp.maximum(0.0, lo - _DELTA)
    hi = jnp.minimum(float(limit - 1), hi + _DELTA)
    return any_p, lo, hi


def _diag_col(row):
    io0 = jax.lax.broadcasted_iota(jnp.int32, (_C, _C), 0)
    io1 = jax.lax.broadcasted_iota(jnp.int32, (_C, _C), 1)
    b = jnp.broadcast_to(row, (_C, _C))
    return jnp.sum(jnp.where(io0 == io1, b, 0.0), axis=1, keepdims=True)


def _stage_kernel(scr_ref, out_ref, bbox_ref, pcol, scol, prow, srow):
    h = pl.program_id(1)

    @pl.when(h == 0)
    def _init():
        pcol[...] = jnp.zeros_like(pcol)
        scol[...] = jnp.zeros_like(scol)

    vals = out_ref[0]
    m = vals[0]
    for c in range(1, _C):
        m = jnp.maximum(m, vals[c])
    mb = jnp.zeros((_HB, _W), jnp.int32)
    for c in range(_C):
        mb = mb | jnp.where(vals[c] == m, jnp.int32(1 << c), 0)
    pm = mb & (-mb)
    sm = jnp.left_shift(jnp.int32(1), scr_ref[0])

    pcol[...] = pcol[...] | pm
    scol[...] = scol[...] | sm
    prow[pl.ds(h * _HB, _HB), 0:1] = _lane_or(pm)
    srow[pl.ds(h * _HB, _HB), 0:1] = _lane_or(sm)

    @pl.when(h == _NH - 1)
    def _finish():
        pcol_red = _sublane_or(pcol[...])
        scol_red = _sublane_or(scol[...])

        csub = jax.lax.broadcasted_iota(jnp.int32, (_C, 1), 0)
        clane = jax.lax.broadcasted_iota(jnp.int32, (1, _C), 1)

        y_x = ((jnp.broadcast_to(pcol_red, (_C, _W)) >> csub) & 1).astype(jnp.float32)
        s_x = ((jnp.broadcast_to(scol_red, (_C, _W)) >> csub) & 1).astype(jnp.float32)
        idx_x = jax.lax.broadcasted_iota(jnp.int32, (1, _W), 1).astype(jnp.float32)
        anyx, x_min, x_max = _axis_stats(s_x, y_x, 1, idx_x, _W)

        y_y = ((jnp.broadcast_to(prow[:, 0:1], (_H, _C)) >> clane) & 1).astype(jnp.float32)
        s_y = ((jnp.broadcast_to(srow[:, 0:1], (_H, _C)) >> clane) & 1).astype(jnp.float32)
        idx_y = jax.lax.broadcasted_iota(jnp.int32, (_H, 1), 0).astype(jnp.float32)
        anyy_r, ylo_r, yhi_r = _axis_stats(s_y, y_y, 0, idx_y, _H)
        anyy = _diag_col(anyy_r)
        y_min = _diag_col(ylo_r)
        y_max = _diag_col(yhi_r)

        keep = anyx * anyy * jnp.where(csub != 0, 1.0, 0.0)
        bbox = jnp.concatenate([x_min, y_min, x_max, y_max], axis=1) * keep
        bbox_ref[0] = bbox


@jax.jit
def _run(scr, out):
    return pl.pallas_call(
        _stage_kernel,
        grid=(_B, _NH),
        in_specs=[
            pl.BlockSpec((1, _HB, _W), lambda b, h: (b, h, 0)),
            pl.BlockSpec((1, _C, _HB, _W), lambda b, h: (b, 0, h, 0)),
        ],
        out_specs=pl.BlockSpec((1, _C, 4), lambda b, h: (b, 0, 0)),
        out_shape=jax.ShapeDtypeStruct((_B, _C, 4), jnp.float32),
        scratch_shapes=[
            pltpu.VMEM((_HB, _W), jnp.int32),
            pltpu.VMEM((_HB, _W), jnp.int32),
            pltpu.VMEM((_H, 128), jnp.int32),
            pltpu.VMEM((_H, 128), jnp.int32),
        ],
        compiler_params=pltpu.CompilerParams(
            dimension_semantics=("arbitrary", "arbitrary"),
        ),
    )(scr, out)


def kernel(scribbles, outputs):
    return _run(scribbles.astype(jnp.int32), outputs)
